# baseline (speedup 1.0000x reference)
import jax
import jax.numpy as jnp
from jax.experimental import pallas as pl
from jax.experimental.pallas import tpu as pltpu

_VOCAB = 100000
_DIM = 64


def _copy_body(x_hbm, o_hbm, sem):
    cp = pltpu.make_async_copy(x_hbm, o_hbm, sem)
    cp.start()
    cp.wait()


def kernel(embeddings):
    return pl.pallas_call(
        _copy_body,
        in_specs=[pl.BlockSpec(memory_space=pl.ANY)],
        out_specs=pl.BlockSpec(memory_space=pl.ANY),
        out_shape=jax.ShapeDtypeStruct((_VOCAB, _DIM), jnp.float32),
        scratch_shapes=[pltpu.SemaphoreType.DMA],
    )(embeddings)


# speedup vs baseline: 15.1159x; 15.1159x over previous
import jax
import jax.numpy as jnp
from jax.experimental import pallas as pl

_VOCAB = 100000
_DIM = 64
_BLOCK = 5000


def _copy_body(x_ref, o_ref):
    o_ref[...] = x_ref[...]


def kernel(embeddings):
    return pl.pallas_call(
        _copy_body,
        grid=(_VOCAB // _BLOCK,),
        in_specs=[pl.BlockSpec((_BLOCK, _DIM), lambda i: (i, 0))],
        out_specs=pl.BlockSpec((_BLOCK, _DIM), lambda i: (i, 0)),
        out_shape=jax.ShapeDtypeStruct((_VOCAB, _DIM), jnp.float32),
    )(embeddings)


# speedup vs baseline: 15.4477x; 1.0219x over previous
import jax
import jax.numpy as jnp
from jax.experimental import pallas as pl

_VOCAB = 100000
_DIM = 64
_BLOCK = 10000


def _copy_body(x_ref, o_ref):
    o_ref[...] = x_ref[...]


def kernel(embeddings):
    return pl.pallas_call(
        _copy_body,
        grid=(_VOCAB // _BLOCK,),
        in_specs=[pl.BlockSpec((_BLOCK, _DIM), lambda i: (i, 0))],
        out_specs=pl.BlockSpec((_BLOCK, _DIM), lambda i: (i, 0)),
        out_shape=jax.ShapeDtypeStruct((_VOCAB, _DIM), jnp.float32),
    )(embeddings)


# speedup vs baseline: 15.5702x; 1.0079x over previous
import jax
import jax.numpy as jnp
from jax.experimental import pallas as pl
from jax.experimental.pallas import tpu as pltpu

_VOCAB = 100000
_DIM = 64
_K = 10
_CH = _VOCAB // _K


def _copy_body(x_hbm, o_hbm, buf, in_sems, out_sems):
    for k in range(_K):
        pltpu.make_async_copy(
            x_hbm.at[pl.ds(k * _CH, _CH), :],
            buf.at[pl.ds(k * _CH, _CH), :],
            in_sems.at[k],
        ).start()
    for k in range(_K):
        pltpu.make_async_copy(
            x_hbm.at[pl.ds(k * _CH, _CH), :],
            buf.at[pl.ds(k * _CH, _CH), :],
            in_sems.at[k],
        ).wait()
        pltpu.make_async_copy(
            buf.at[pl.ds(k * _CH, _CH), :],
            o_hbm.at[pl.ds(k * _CH, _CH), :],
            out_sems.at[k],
        ).start()
    for k in range(_K):
        pltpu.make_async_copy(
            buf.at[pl.ds(k * _CH, _CH), :],
            o_hbm.at[pl.ds(k * _CH, _CH), :],
            out_sems.at[k],
        ).wait()


def kernel(embeddings):
    return pl.pallas_call(
        _copy_body,
        in_specs=[pl.BlockSpec(memory_space=pl.ANY)],
        out_specs=pl.BlockSpec(memory_space=pl.ANY),
        out_shape=jax.ShapeDtypeStruct((_VOCAB, _DIM), jnp.float32),
        scratch_shapes=[
            pltpu.VMEM((_VOCAB, _DIM), jnp.float32),
            pltpu.SemaphoreType.DMA((_K,)),
            pltpu.SemaphoreType.DMA((_K,)),
        ],
    )(embeddings)
